# baseline (speedup 1.0000x reference)
import functools

import jax
import jax.numpy as jnp
from jax import lax
from jax.experimental import pallas as pl
from jax.experimental.pallas import tpu as pltpu
from jax.experimental.pallas import tpu_sc as plsc

N = 10000
E = 100000
D = 768
NC, NS = 2, 16
CH = 64
NCH = D // CH
BATCH = 128
NB = 52
RING = 4
E_PAD = NS * NB * BATCH
N_PAD = 10240
RPS = N_PAD // NS
BN = 400


def _mm_body(x_ref, w_ref, y_ref):
    res = jnp.dot(x_ref[...], w_ref[0],
                  preferred_element_type=jnp.float32)
    for j in range(NCH):
        y_ref[0, j] = res[:, j * CH:(j + 1) * CH]


def _matmul_chunked(x, Wstack):
    return pl.pallas_call(
        _mm_body,
        grid=(NC, N // BN),
        in_specs=[
            pl.BlockSpec((BN, D), lambda d, i: (i, 0)),
            pl.BlockSpec((1, D, D), lambda d, i: (d, 0, 0)),
        ],
        out_specs=pl.BlockSpec((1, NCH, BN, CH), lambda d, i: (d, 0, i, 0)),
        out_shape=jax.ShapeDtypeStruct((NC, NCH, N, CH), jnp.float32),
    )(x, Wstack)


def _combine_body(y_ref, agg_ref, deg_ref, b_ref, o_ref):
    deg = deg_ref[0][:, 0:1]
    mask = (deg == 0.0).astype(jnp.float32)
    y = jnp.concatenate([y_ref[0, 0], y_ref[0, 1]], axis=1)
    agg = jnp.concatenate([agg_ref[0, 0], agg_ref[0, 1]], axis=1)
    o_ref[...] = jnp.maximum(y * mask + agg + b_ref[0], 0.0)


def _combine(d, y_t, agg_t, deg, bvec):
    return pl.pallas_call(
        _combine_body,
        grid=(N // BN, NCH // 2),
        in_specs=[
            pl.BlockSpec((1, 2, BN, CH), lambda i, c, d=d: (d, c, i, 0)),
            pl.BlockSpec((1, 2, BN, CH), lambda i, c, d=d: (d, c, i, 0)),
            pl.BlockSpec((1, BN, CH), lambda i, c, d=d: (d, i, 0)),
            pl.BlockSpec((1, 1, 2 * CH), lambda i, c: (c, 0, 0)),
        ],
        out_specs=pl.BlockSpec((BN, 2 * CH), lambda i, c: (i, c)),
        out_shape=jax.ShapeDtypeStruct((N, D), jnp.float32),
    )(y_t, agg_t, deg, bvec)


_MESH = plsc.VectorSubcoreMesh(core_axis_name="c", subcore_axis_name="s",
                               num_cores=NC, num_subcores=NS)


@functools.partial(
    pl.kernel,
    out_type=[jax.ShapeDtypeStruct((NC, NCH, N_PAD, CH), jnp.float32),
              jax.ShapeDtypeStruct((NC, N_PAD, CH), jnp.float32)],
    mesh=_MESH,
    compiler_params=pltpu.CompilerParams(use_tc_tiling_on_sc=False),
    scratch_types=[
        pltpu.VMEM((NB, BATCH), jnp.int32),
        pltpu.VMEM((NB, BATCH), jnp.int32),
        pltpu.VMEM((NB, BATCH), jnp.int32),
        pltpu.VMEM((RING, BATCH, CH), jnp.float32),
        pltpu.VMEM((BATCH, CH), jnp.float32),
        pltpu.VMEM((BATCH, CH), jnp.float32),
        pltpu.VMEM_SHARED((N_PAD, CH), jnp.float32),
        [pltpu.SemaphoreType.DMA] * RING,
        [pltpu.SemaphoreType.DMA] * RING,
    ],
)
def _sc_agg(y_hbm, g_hbm, s_hbm, dg_hbm, agg_hbm, deg_hbm,
            gidx_v, sidx_v, didx_v, rows_v, zt_v, ot_v, acc_sh, gsem, ssem):
    cid = lax.axis_index("c")
    sid = lax.axis_index("s")

    @pl.loop(0, BATCH)
    def _(i):
        @pl.loop(0, CH, step=16)
        def _(j):
            zt_v[i, pl.ds(j, 16)] = jnp.zeros((16,), jnp.float32)
            ot_v[i, pl.ds(j, 16)] = jnp.ones((16,), jnp.float32)

    def zero_acc():
        @pl.loop(0, RPS, step=BATCH)
        def _(j):
            pltpu.sync_copy(zt_v, acc_sh.at[pl.ds(sid * RPS + j, BATCH)])
        plsc.subcore_barrier()

    def run_dir(y_dir, g_dir, s_dir, dg_dir, agg_dir, deg_dir):
        pltpu.sync_copy(g_dir.at[sid], gidx_v)
        pltpu.sync_copy(s_dir.at[sid], sidx_v)
        pltpu.sync_copy(dg_dir.at[sid], didx_v)

        zero_acc()

        @pl.loop(0, NB)
        def _(b):
            pltpu.sync_copy(ot_v, acc_sh.at[didx_v.at[b]], add=True)
        plsc.subcore_barrier()

        pltpu.sync_copy(acc_sh.at[pl.ds(sid * RPS, RPS)],
                        deg_dir.at[pl.ds(sid * RPS, RPS)])

        @pl.loop(0, NCH)
        def _(c):
            zero_acc()
            yc = y_dir.at[c]

            def gather_start(b, s):
                pltpu.async_copy(yc.at[gidx_v.at[b]], rows_v.at[s], gsem[s])

            def gather_wait(b, s):
                pltpu.make_async_copy(yc.at[gidx_v.at[b]], rows_v.at[s],
                                      gsem[s]).wait()

            for s in range(RING):
                gather_start(s, s)

            @pl.loop(0, NB - RING, step=RING)
            def _(b):
                for s in range(RING):
                    gather_wait(b + s, s)
                    pltpu.sync_copy(rows_v.at[s],
                                    acc_sh.at[sidx_v.at[b + s]], add=True)
                    gather_start(b + RING + s, s)

            for s in range(RING):
                gather_wait(NB - RING + s, s)
                pltpu.sync_copy(rows_v.at[s],
                                acc_sh.at[sidx_v.at[NB - RING + s]],
                                add=True)
            plsc.subcore_barrier()

            pltpu.sync_copy(acc_sh.at[pl.ds(sid * RPS, RPS)],
                            agg_dir.at[c].at[pl.ds(sid * RPS, RPS)])

    @pl.when(cid == 0)
    def _():
        run_dir(y_hbm.at[0], g_hbm.at[0], s_hbm.at[0], dg_hbm.at[0],
                agg_hbm.at[0], deg_hbm.at[0])

    @pl.when(cid == 1)
    def _():
        run_dir(y_hbm.at[1], g_hbm.at[1], s_hbm.at[1], dg_hbm.at[1],
                agg_hbm.at[1], deg_hbm.at[1])


def kernel(x, edge_index, sources, destinations, W_root, b_root, W_leaf,
           b_leaf):
    pad = E_PAD - E
    dummy = (N + (jnp.arange(pad, dtype=jnp.int32) % (N_PAD - N)))
    zpad = jnp.zeros((pad,), jnp.int32)
    shard = (NS, NB, BATCH)
    src0 = jnp.concatenate([sources, zpad]).reshape(shard)
    srcd = jnp.concatenate([sources, dummy]).reshape(shard)
    dst0 = jnp.concatenate([destinations, zpad]).reshape(shard)
    dstd = jnp.concatenate([destinations, dummy]).reshape(shard)

    G = jnp.stack([src0, dst0])
    S = jnp.stack([dstd, srcd])
    DG = jnp.stack([srcd, dstd])
    Wstack = jnp.stack([W_leaf, W_root])

    Y = _matmul_chunked(x, Wstack)
    AGG, DEG = _sc_agg(Y, G, S, DG)

    bh = NCH // 2
    leaf_emb = _combine(0, Y, AGG, DEG, b_leaf.reshape(bh, 1, 2 * CH))
    root_emb = _combine(1, Y, AGG, DEG, b_root.reshape(bh, 1, 2 * CH))
    return (root_emb, leaf_emb)


# speedup vs baseline: 2.6262x; 2.6262x over previous
import functools

import jax
import jax.numpy as jnp
from jax import lax
from jax.experimental import pallas as pl
from jax.experimental.pallas import tpu as pltpu
from jax.experimental.pallas import tpu_sc as plsc

N = 10000
E = 100000
D = 768
NC, NS = 2, 16
CH = 64
NCH = D // CH
BATCH = 128
NB = 49
RING = 4
E_PAD = NS * NB * BATCH
N_PAD = 10240
RPS = N_PAD // NS
BN = 400


def _mm_body(x_ref, w_ref, y_ref):
    res = jnp.dot(x_ref[...], w_ref[0],
                  preferred_element_type=jnp.float32)
    for j in range(NCH):
        y_ref[0, j] = res[:, j * CH:(j + 1) * CH]


def _matmul_chunked(x, Wstack):
    return pl.pallas_call(
        _mm_body,
        grid=(NC, N // BN),
        in_specs=[
            pl.BlockSpec((BN, D), lambda d, i: (i, 0)),
            pl.BlockSpec((1, D, D), lambda d, i: (d, 0, 0)),
        ],
        out_specs=pl.BlockSpec((1, NCH, BN, CH), lambda d, i: (d, 0, i, 0)),
        out_shape=jax.ShapeDtypeStruct((NC, NCH, N, CH), jnp.float32),
    )(x, Wstack)


def _combine_body(y_ref, agg_ref, deg_ref, b_ref, o_ref):
    deg = deg_ref[0][:, 0:1]
    mask = (deg == 0.0).astype(jnp.float32)
    y = jnp.concatenate([y_ref[0, 0], y_ref[0, 1]], axis=1)
    agg = jnp.concatenate([agg_ref[0, 0], agg_ref[0, 1]], axis=1)
    o_ref[...] = jnp.maximum(y * mask + agg + b_ref[0], 0.0)


def _combine(d, y_t, agg_t, deg, bvec):
    return pl.pallas_call(
        _combine_body,
        grid=(N // BN, NCH // 2),
        in_specs=[
            pl.BlockSpec((1, 2, BN, CH), lambda i, c, d=d: (d, c, i, 0)),
            pl.BlockSpec((1, 2, BN, CH), lambda i, c, d=d: (d, c, i, 0)),
            pl.BlockSpec((1, BN, CH), lambda i, c, d=d: (d, i, 0)),
            pl.BlockSpec((1, 1, 2 * CH), lambda i, c: (c, 0, 0)),
        ],
        out_specs=pl.BlockSpec((BN, 2 * CH), lambda i, c: (i, c)),
        out_shape=jax.ShapeDtypeStruct((N, D), jnp.float32),
    )(y_t, agg_t, deg, bvec)


_MESH = plsc.VectorSubcoreMesh(core_axis_name="c", subcore_axis_name="s",
                               num_cores=NC, num_subcores=NS)


@functools.partial(
    pl.kernel,
    out_type=[jax.ShapeDtypeStruct((NC, NCH, N_PAD, CH), jnp.float32),
              jax.ShapeDtypeStruct((NC, N_PAD, CH), jnp.float32)],
    mesh=_MESH,
    compiler_params=pltpu.CompilerParams(use_tc_tiling_on_sc=False),
    scratch_types=[
        pltpu.VMEM((NB, BATCH), jnp.int32),
        pltpu.VMEM((NB, BATCH), jnp.int32),
        pltpu.VMEM((NB, BATCH), jnp.int32),
        pltpu.VMEM((RING, BATCH, CH), jnp.float32),
        pltpu.VMEM((BATCH, CH), jnp.float32),
        pltpu.VMEM((BATCH, CH), jnp.float32),
        pltpu.VMEM_SHARED((N_PAD, CH), jnp.float32),
        [pltpu.SemaphoreType.DMA] * RING,
        [pltpu.SemaphoreType.DMA] * RING,
    ],
)
def _sc_agg(y_hbm, g_hbm, s_hbm, dg_hbm, agg_hbm, deg_hbm,
            gidx_v, sidx_v, didx_v, rows_v, zt_v, ot_v, acc_sh, gsem, ssem):
    cid = lax.axis_index("c")
    sid = lax.axis_index("s")

    @pl.loop(0, BATCH)
    def _(i):
        @pl.loop(0, CH, step=16)
        def _(j):
            zt_v[i, pl.ds(j, 16)] = jnp.zeros((16,), jnp.float32)
            ot_v[i, pl.ds(j, 16)] = jnp.ones((16,), jnp.float32)

    def zero_acc():
        @pl.loop(0, RPS, step=BATCH)
        def _(j):
            pltpu.sync_copy(zt_v, acc_sh.at[pl.ds(sid * RPS + j, BATCH)])
        plsc.subcore_barrier()

    def run_dir(y_dir, g_dir, s_dir, dg_dir, agg_dir, deg_dir):
        pltpu.sync_copy(g_dir.at[sid], gidx_v)
        pltpu.sync_copy(s_dir.at[sid], sidx_v)
        pltpu.sync_copy(dg_dir.at[sid], didx_v)

        zero_acc()

        @pl.loop(0, NB)
        def _(b):
            pltpu.sync_copy(ot_v, acc_sh.at[didx_v.at[b]], add=True)
        plsc.subcore_barrier()

        pltpu.sync_copy(acc_sh.at[pl.ds(sid * RPS, RPS)],
                        deg_dir.at[pl.ds(sid * RPS, RPS)])

        @pl.loop(0, NCH)
        def _(c):
            zero_acc()
            yc = y_dir.at[c]

            def gather_start(b, s):
                pltpu.async_copy(yc.at[gidx_v.at[b]], rows_v.at[s], gsem[s])

            def gather_wait(b, s):
                pltpu.make_async_copy(yc.at[gidx_v.at[b]], rows_v.at[s],
                                      gsem[s]).wait()

            for s in range(RING):
                gather_start(s, s)

            @pl.loop(0, NB - RING - 1, step=RING)
            def _(b):
                for s in range(RING):
                    gather_wait(b + s, s)
                    pltpu.sync_copy(rows_v.at[s],
                                    acc_sh.at[sidx_v.at[b + s]], add=True)
                    gather_start(b + RING + s, s)

            e = NB - RING - 1
            for s in range(RING):
                gather_wait(e + s, s)
                pltpu.sync_copy(rows_v.at[s],
                                acc_sh.at[sidx_v.at[e + s]], add=True)
            gather_start(NB - 1, 0)
            gather_wait(NB - 1, 0)
            pltpu.sync_copy(rows_v.at[0], acc_sh.at[sidx_v.at[NB - 1]],
                            add=True)
            plsc.subcore_barrier()

            pltpu.sync_copy(acc_sh.at[pl.ds(sid * RPS, RPS)],
                            agg_dir.at[c].at[pl.ds(sid * RPS, RPS)])

    @pl.when(cid == 0)
    def _():
        run_dir(y_hbm.at[0], g_hbm.at[0], s_hbm.at[0], dg_hbm.at[0],
                agg_hbm.at[0], deg_hbm.at[0])

    @pl.when(cid == 1)
    def _():
        run_dir(y_hbm.at[1], g_hbm.at[1], s_hbm.at[1], dg_hbm.at[1],
                agg_hbm.at[1], deg_hbm.at[1])


def kernel(x, edge_index, sources, destinations, W_root, b_root, W_leaf,
           b_leaf):
    pps = E_PAD // NS - E // NS
    dummy = (N + (jnp.arange(NS * pps, dtype=jnp.int32) % (N_PAD - N))
             ).reshape(NS, pps)
    zpad = jnp.zeros((NS, pps), jnp.int32)
    shard = (NS, NB, BATCH)

    def shard_pad(a, padv):
        return jnp.concatenate([a.reshape(NS, E // NS), padv],
                               axis=1).reshape(shard)

    src0 = shard_pad(sources, zpad)
    srcd = shard_pad(sources, dummy)
    dst0 = shard_pad(destinations, zpad)
    dstd = shard_pad(destinations, dummy)

    G = jnp.stack([src0, dst0])
    S = jnp.stack([dstd, srcd])
    DG = jnp.stack([srcd, dstd])
    Wstack = jnp.stack([W_leaf, W_root])

    Y = _matmul_chunked(x, Wstack)
    AGG, DEG = _sc_agg(Y, G, S, DG)

    bh = NCH // 2
    leaf_emb = _combine(0, Y, AGG, DEG, b_leaf.reshape(bh, 1, 2 * CH))
    root_emb = _combine(1, Y, AGG, DEG, b_root.reshape(bh, 1, 2 * CH))
    return (root_emb, leaf_emb)
